# baseline (speedup 1.0000x reference)
import jax
import jax.numpy as jnp
from jax.experimental import pallas as pl
from jax.experimental.pallas import tpu as pltpu

R = 20000
D = 1024
K = 117
BR = 1000
NBUF = 4
NSTEP = R // BR


def _mm_kernel(x_hbm, wt_ref, b_ref, o_hbm, xbuf, obuf, insem, outsem):
    i = pl.program_id(0)

    def in_copy(step, buf):
        return pltpu.make_async_copy(
            x_hbm.at[pl.ds(step * BR, BR), :], xbuf.at[buf], insem.at[buf]
        )

    def out_copy(step, ob):
        return pltpu.make_async_copy(
            obuf.at[ob], o_hbm.at[pl.ds(step * BR, BR), :], outsem.at[ob]
        )

    @pl.when(i == 0)
    def _prologue():
        for j in range(NBUF):
            in_copy(j, j).start()

    buf = jax.lax.rem(i, NBUF)
    in_copy(i, buf).wait()
    acc = jax.lax.dot_general(
        xbuf[buf], wt_ref[...],
        dimension_numbers=(((1,), (0,)), ((), ())),
        preferred_element_type=jnp.float32,
    )

    ob = jax.lax.rem(i, 2)

    @pl.when(i >= 2)
    def _wait_prev_store():
        out_copy(i - 2, ob).wait()

    obuf[ob] = acc + b_ref[...]
    out_copy(i, ob).start()

    @pl.when(i + NBUF < NSTEP)
    def _refill():
        in_copy(i + NBUF, buf).start()

    @pl.when(i == NSTEP - 1)
    def _drain():
        out_copy(i - 1, jax.lax.rem(i - 1, 2)).wait()
        out_copy(i, ob).wait()


def kernel(x, W, b):
    wt = W.T
    bp = b.reshape(1, K)
    return pl.pallas_call(
        _mm_kernel,
        grid=(NSTEP,),
        in_specs=[
            pl.BlockSpec(memory_space=pl.ANY),
            pl.BlockSpec((D, K), lambda i: (0, 0)),
            pl.BlockSpec((1, K), lambda i: (0, 0)),
        ],
        out_specs=pl.BlockSpec(memory_space=pl.ANY),
        out_shape=jax.ShapeDtypeStruct((R, K), jnp.float32),
        scratch_shapes=[
            pltpu.VMEM((NBUF, BR, D), jnp.float32),
            pltpu.VMEM((2, BR, K), jnp.float32),
            pltpu.SemaphoreType.DMA((NBUF,)),
            pltpu.SemaphoreType.DMA((2,)),
        ],
        compiler_params=pltpu.CompilerParams(
            dimension_semantics=("arbitrary",),
        ),
    )(x, wt, bp)
